# Initial kernel scaffold; baseline (speedup 1.0000x reference)
import jax
import jax.numpy as jnp
from jax.experimental import pallas as pl


def kernel(x, scale, bias):
    raise NotImplementedError("write your pallas kernel here")


# speedup vs baseline: 1.5940x; 1.5940x over previous
import functools
import math

import jax
import jax.numpy as jnp
from jax.experimental import pallas as pl
from jax.experimental.pallas import tpu as pltpu

_WINDOW = 256
_TAU_INTER = 0.5
_TAU_TEMP = 1.2
_MAX_AGE = 4


def _body(x_ref, scale_ref, bias_ref, o_ref, sig_ref, age_ref):
    w = pl.program_id(0)
    xw = x_ref[0]
    tw, b, d = xw.shape

    sig = jnp.mean(xw, axis=0)
    diff = jnp.abs(xw[1:] - xw[:-1])
    vtemp = jnp.sum(diff, axis=(0, 2), keepdims=False) * (1.0 / ((tw - 1) * d))
    vtemp = vtemp.reshape(b, 1)

    prev_sig = sig_ref[...]
    delta = sig - prev_sig
    d2 = jnp.sum(delta * delta, axis=1, keepdims=True)
    d_inter = jnp.sqrt(d2) * (1.0 / math.sqrt(d))

    age = age_ref[...]
    refresh = (
        (w == 0)
        | (age >= _MAX_AGE)
        | (d_inter > _TAU_INTER)
        | (vtemp > _TAU_TEMP)
    )

    new_sig = jnp.where(refresh, sig, prev_sig)
    sig_ref[...] = new_sig
    age_ref[...] = jnp.where(refresh, 0, age + 1)

    scale = scale_ref[0]
    bias = bias_ref[0]
    y_full = xw * scale + bias
    y_reuse = new_sig * scale + bias
    o_ref[0] = jnp.where(refresh[None], y_full, y_reuse[None])


def kernel(x, scale, bias):
    t, b, d = x.shape
    nw = t // _WINDOW
    xr = x.reshape(nw, _WINDOW, b, d)
    out = pl.pallas_call(
        _body,
        grid=(nw,),
        in_specs=[
            pl.BlockSpec((1, _WINDOW, b, d), lambda w: (w, 0, 0, 0)),
            pl.BlockSpec((1, d), lambda w: (0, 0)),
            pl.BlockSpec((1, d), lambda w: (0, 0)),
        ],
        out_specs=pl.BlockSpec((1, _WINDOW, b, d), lambda w: (w, 0, 0, 0)),
        out_shape=jax.ShapeDtypeStruct((nw, _WINDOW, b, d), x.dtype),
        scratch_shapes=[
            pltpu.VMEM((b, d), jnp.float32),
            pltpu.VMEM((b, 1), jnp.int32),
        ],
    )(xr, scale.reshape(1, d), bias.reshape(1, d))
    return out.reshape(t, b, d)
